# baseline (speedup 1.0000x reference)
import functools

import jax
import jax.numpy as jnp
from jax.experimental import pallas as pl
from jax.experimental.pallas import tpu as pltpu
from jax.experimental.pallas import tpu_sc as plsc

Z_DIM = 64


def _lookup_body(z_hbm, idx_hbm, out_hbm, idx_s):
    pltpu.sync_copy(idx_hbm, idx_s)
    r = idx_s[0]
    pltpu.sync_copy(z_hbm.at[r], out_hbm)


_lookup = functools.partial(
    pl.kernel,
    out_type=jax.ShapeDtypeStruct((Z_DIM,), jnp.float32),
    mesh=plsc.ScalarSubcoreMesh(axis_name="c", num_cores=1),
    scratch_types=[
        pltpu.SMEM((1,), jnp.int32),
    ],
)(_lookup_body)


def kernel(Z, list_index):
    idx = jnp.asarray(list_index, jnp.int32).reshape((1,))
    return _lookup(Z, idx)
